# baseline (speedup 1.0000x reference)
import functools

import jax
import jax.numpy as jnp
from jax import lax
from jax.experimental import pallas as pl
from jax.experimental.pallas import tpu as pltpu
from jax.experimental.pallas import tpu_sc as plsc

NC = 2
NS = 16
NW = NC * NS
L = 16
CH = 128
NBUF = 5
LAG = 2


@functools.partial(jax.jit, static_argnums=(2, 3))
def _gather(weight, idxp, nch, dim):
    bpw = nch * CH
    ngrp = nch // NBUF

    mesh = plsc.VectorSubcoreMesh(core_axis_name="c", subcore_axis_name="s")

    @functools.partial(
        pl.kernel,
        mesh=mesh,
        out_type=jax.ShapeDtypeStruct((NW * bpw, dim), jnp.float32),
        scratch_types=[
            pltpu.VMEM((nch, CH // 2), jnp.int32),
            pltpu.VMEM((NBUF, CH), jnp.int32),
            pltpu.VMEM((NBUF, CH, dim), jnp.float32),
            pltpu.SemaphoreType.DMA,
            pltpu.SemaphoreType.DMA,
        ],
    )
    def body(table_hbm, idx_hbm, out_hbm, idxp_v, idx_v, rows_v, gsem, ssem):
        c = lax.axis_index("c")
        s = lax.axis_index("s")
        wid = s * NC + c
        base = wid * bpw

        pltpu.sync_copy(idx_hbm.at[wid], idxp_v)

        def prep(j, b):
            for g in range(CH // (2 * L)):
                p = idxp_v[j, pl.ds(g * L, L)]
                idx_v[b, pl.ds(g * 2 * L, L)] = (p & 0xFFFF) + 1
                idx_v[b, pl.ds(g * 2 * L + L, L)] = lax.shift_right_logical(p, 16) + 1

        def start_gather(b):
            pltpu.async_copy(table_hbm.at[idx_v.at[b]], rows_v.at[b], gsem)

        def start_scatter(j, b):
            pltpu.async_copy(rows_v.at[b], out_hbm.at[pl.ds(base + j * CH, CH)], ssem)

        def wait_gather(b):
            pltpu.make_async_copy(table_hbm.at[pl.ds(0, CH)], rows_v.at[b], gsem).wait()

        def wait_scatter(b):
            pltpu.make_async_copy(rows_v.at[b], out_hbm.at[pl.ds(0, CH)], ssem).wait()

        for b in range(NBUF):
            prep(b, b)
            start_gather(b)
            if b >= LAG:
                wait_gather(b - LAG)
                start_scatter(b - LAG, b - LAG)

        def group(g, carry):
            for b in range(NBUF):
                j = g * NBUF + b
                prep(j, b)
                wait_scatter(b)
                start_gather(b)
                bp = (b - LAG) % NBUF
                wait_gather(bp)
                start_scatter(j - LAG, bp)
            return carry

        lax.fori_loop(1, ngrp, group, 0)

        for t in range(LAG):
            jj = nch - LAG + t
            b = jj % NBUF
            wait_gather(b)
            start_scatter(jj, b)
        for t in range(NBUF):
            wait_scatter(t)

    return body(weight, idxp)


def kernel(input_positions, weight):
    bsz, slen = input_positions.shape
    dim = weight.shape[1]
    total = bsz * slen
    nch = total // (NW * CH)
    r = input_positions.astype(jnp.int32).reshape(NW, nch, CH // (2 * L), 2, L)
    idxp = (r[:, :, :, 0, :] | (r[:, :, :, 1, :] << 16)).reshape(NW, nch, CH // 2)
    out = _gather(weight, idxp, nch, dim)
    return out.reshape(bsz, slen, dim)


# speedup vs baseline: 1.3445x; 1.3445x over previous
import functools

import jax
import jax.numpy as jnp
from jax import lax
from jax.experimental import pallas as pl
from jax.experimental.pallas import tpu as pltpu
from jax.experimental.pallas import tpu_sc as plsc

NC = 2
NS = 16
NW = NC * NS
L = 16
CH = 128
GRP = 4


@functools.partial(jax.jit, static_argnums=(2, 3))
def _gather(weight, idx, nch, dim):
    bpw = nch * CH
    ngrp = nch // GRP

    mesh = plsc.VectorSubcoreMesh(core_axis_name="c", subcore_axis_name="s")

    @functools.partial(
        pl.kernel,
        mesh=mesh,
        out_type=jax.ShapeDtypeStruct((NW * bpw, dim), jnp.float32),
        scratch_types=[
            pltpu.VMEM((nch, CH), jnp.int32),
            pltpu.VMEM((GRP, CH, dim), jnp.float32),
            pltpu.SemaphoreType.DMA,
            pltpu.SemaphoreType.DMA,
        ],
    )
    def body(table_hbm, idx_hbm, out_hbm, idx_v, rows_v, gsem, ssem):
        c = lax.axis_index("c")
        s = lax.axis_index("s")
        wid = s * NC + c
        base = wid * bpw

        pltpu.sync_copy(idx_hbm.at[wid], idx_v)

        def plus1(i, carry):
            for g in range(CH // L):
                idx_v[i, pl.ds(g * L, L)] = idx_v[i, pl.ds(g * L, L)] + 1
            return carry

        lax.fori_loop(0, nch, plus1, 0)

        def start_gather(j, b):
            pltpu.async_copy(table_hbm.at[idx_v.at[j]], rows_v.at[b], gsem)

        def start_scatter(j, b):
            pltpu.async_copy(rows_v.at[b], out_hbm.at[pl.ds(base + j * CH, CH)], ssem)

        def wait_gather(b):
            pltpu.make_async_copy(table_hbm.at[pl.ds(0, CH)], rows_v.at[b], gsem).wait()

        def wait_scatter(b):
            pltpu.make_async_copy(rows_v.at[b], out_hbm.at[pl.ds(0, CH)], ssem).wait()

        for b in range(GRP):
            start_gather(b, b)
        for b in range(GRP):
            wait_gather(b)
            start_scatter(b, b)

        def group(g, carry):
            j0 = g * GRP
            for b in range(GRP):
                wait_scatter(b)
                start_gather(j0 + b, b)
            for b in range(GRP):
                wait_gather(b)
                start_scatter(j0 + b, b)
            return carry

        lax.fori_loop(1, ngrp, group, 0)

        for b in range(GRP):
            wait_scatter(b)

    return body(weight, idx)


def kernel(input_positions, weight):
    bsz, slen = input_positions.shape
    dim = weight.shape[1]
    total = bsz * slen
    nch = total // (NW * CH)
    idx = input_positions.astype(jnp.int32).reshape(NW, nch, CH)
    out = _gather(weight, idx, nch, dim)
    return out.reshape(bsz, slen, dim)


# speedup vs baseline: 2.3202x; 1.7258x over previous
import functools

import jax
import jax.numpy as jnp
from jax import lax
from jax.experimental import pallas as pl
from jax.experimental.pallas import tpu as pltpu
from jax.experimental.pallas import tpu_sc as plsc

NC = 2
NS = 16
NW = NC * NS
CH = 128
GRP = 2
V = 8192


@functools.partial(jax.jit, static_argnums=(2, 3))
def _gather(weight, idx, nch, dim):
    bpw = nch * CH
    ngrp = nch // GRP
    rpt = V // NS

    mesh = plsc.VectorSubcoreMesh(core_axis_name="c", subcore_axis_name="s")

    @functools.partial(
        pl.kernel,
        mesh=mesh,
        out_type=jax.ShapeDtypeStruct((NW * bpw, dim), jnp.float32),
        scratch_types=[
            pltpu.VMEM((nch, CH), jnp.int32),
            pltpu.VMEM((GRP * CH, dim), jnp.float32),
            pltpu.VMEM_SHARED((V, dim), jnp.float32),
            pltpu.SemaphoreType.DMA,
            pltpu.SemaphoreType.DMA,
        ],
    )
    def body(table_hbm, idx_hbm, out_hbm, idx_v, rows_v, tab_s, gsem, ssem):
        c = lax.axis_index("c")
        s = lax.axis_index("s")
        wid = s * NC + c
        base = wid * bpw

        for t in range(rpt // CH):
            b = (t % GRP) * CH
            src = (s * (rpt // CH) + t) * CH
            pltpu.sync_copy(table_hbm.at[pl.ds(src, CH)], rows_v.at[pl.ds(b, CH)])
            if t == 0:

                @pl.when(s == 0)
                def _first_chunk():
                    pltpu.sync_copy(rows_v.at[pl.ds(1, CH - 1)], tab_s.at[pl.ds(0, CH - 1)])

                @pl.when(s != 0)
                def _chunk0():
                    pltpu.sync_copy(rows_v.at[pl.ds(b, CH)], tab_s.at[pl.ds(src - 1, CH)])

            else:
                pltpu.sync_copy(rows_v.at[pl.ds(b, CH)], tab_s.at[pl.ds(src - 1, CH)])

        @pl.when(s == NS - 1)
        def _last_row():
            pltpu.sync_copy(table_hbm.at[pl.ds(V, 1)], rows_v.at[pl.ds(0, 1)])
            pltpu.sync_copy(rows_v.at[pl.ds(0, 1)], tab_s.at[pl.ds(V - 1, 1)])

        pltpu.sync_copy(idx_hbm.at[wid], idx_v)

        plsc.subcore_barrier()

        def start_gather(j, b):
            pltpu.async_copy(tab_s.at[idx_v.at[j]], rows_v.at[pl.ds(b * CH, CH)], gsem)

        def start_scatter(j, b):
            pltpu.async_copy(
                rows_v.at[pl.ds(b * CH, CH)], out_hbm.at[pl.ds(base + j * CH, CH)], ssem
            )

        def wait_gather(b):
            pltpu.make_async_copy(
                tab_s.at[pl.ds(0, CH)], rows_v.at[pl.ds(b * CH, CH)], gsem
            ).wait()

        def wait_scatter(b):
            pltpu.make_async_copy(
                rows_v.at[pl.ds(b * CH, CH)], out_hbm.at[pl.ds(0, CH)], ssem
            ).wait()

        for b in range(GRP):
            start_gather(b, b)
        for b in range(GRP):
            wait_gather(b)
            start_scatter(b, b)

        def group(g, carry):
            j0 = g * GRP
            for b in range(GRP):
                wait_scatter(b)
                start_gather(j0 + b, b)
            for b in range(GRP):
                wait_gather(b)
                start_scatter(j0 + b, b)
            return carry

        lax.fori_loop(1, ngrp, group, 0)

        for b in range(GRP):
            wait_scatter(b)

    return body(weight, idx)


def kernel(input_positions, weight):
    bsz, slen = input_positions.shape
    dim = weight.shape[1]
    total = bsz * slen
    nch = total // (NW * CH)
    idx = input_positions.astype(jnp.int32).reshape(NW, nch, CH)
    out = _gather(weight, idx, nch, dim)
    return out.reshape(bsz, slen, dim)


# speedup vs baseline: 2.3506x; 1.0131x over previous
import functools

import jax
import jax.numpy as jnp
from jax import lax
from jax.experimental import pallas as pl
from jax.experimental.pallas import tpu as pltpu
from jax.experimental.pallas import tpu_sc as plsc

NC = 2
NS = 16
NW = NC * NS
CH = 128
GRP = 2
V = 8192


@functools.partial(jax.jit, static_argnums=(2, 3))
def _gather(weight, idx, nch, dim):
    bpw = nch * CH
    ngrp = nch // GRP
    rpt = V // NS

    mesh = plsc.VectorSubcoreMesh(core_axis_name="c", subcore_axis_name="s")

    @functools.partial(
        pl.kernel,
        mesh=mesh,
        out_type=jax.ShapeDtypeStruct((NW * bpw, dim), jnp.float32),
        scratch_types=[
            pltpu.VMEM((nch, CH), jnp.int32),
            pltpu.VMEM((GRP * CH, dim), jnp.float32),
            pltpu.VMEM_SHARED((V, dim), jnp.float32),
            pltpu.SemaphoreType.DMA,
            pltpu.SemaphoreType.DMA,
        ],
    )
    def body(table_hbm, idx_hbm, out_hbm, idx_v, rows_v, tab_s, gsem, ssem):
        c = lax.axis_index("c")
        s = lax.axis_index("s")
        wid = s * NC + c
        base = wid * bpw

        src0 = s * rpt

        @pl.when(s == 0)
        def _first_stripe():
            pltpu.sync_copy(table_hbm.at[pl.ds(8, rpt - 8)], tab_s.at[pl.ds(7, rpt - 8)])
            pltpu.sync_copy(table_hbm.at[pl.ds(0, 8)], rows_v.at[pl.ds(0, 8)])
            pltpu.sync_copy(rows_v.at[pl.ds(1, 7)], tab_s.at[pl.ds(0, 7)])

        @pl.when(s != 0)
        def _stripe():
            pltpu.sync_copy(table_hbm.at[pl.ds(src0, rpt)], tab_s.at[pl.ds(src0 - 1, rpt)])

        @pl.when(s == NS - 1)
        def _last_row():
            pltpu.sync_copy(table_hbm.at[pl.ds(V, 1)], rows_v.at[pl.ds(0, 1)])
            pltpu.sync_copy(rows_v.at[pl.ds(0, 1)], tab_s.at[pl.ds(V - 1, 1)])

        pltpu.sync_copy(idx_hbm.at[wid], idx_v)

        plsc.subcore_barrier()

        def start_gather(j, b):
            pltpu.async_copy(tab_s.at[idx_v.at[j]], rows_v.at[pl.ds(b * CH, CH)], gsem)

        def start_scatter(j, b):
            pltpu.async_copy(
                rows_v.at[pl.ds(b * CH, CH)], out_hbm.at[pl.ds(base + j * CH, CH)], ssem
            )

        def wait_gather(b):
            pltpu.make_async_copy(
                tab_s.at[pl.ds(0, CH)], rows_v.at[pl.ds(b * CH, CH)], gsem
            ).wait()

        def wait_scatter(b):
            pltpu.make_async_copy(
                rows_v.at[pl.ds(b * CH, CH)], out_hbm.at[pl.ds(0, CH)], ssem
            ).wait()

        for b in range(GRP):
            start_gather(b, b)
        for b in range(GRP):
            wait_gather(b)
            start_scatter(b, b)

        def group(g, carry):
            j0 = g * GRP
            for b in range(GRP):
                wait_scatter(b)
                start_gather(j0 + b, b)
            for b in range(GRP):
                wait_gather(b)
                start_scatter(j0 + b, b)
            return carry

        lax.fori_loop(1, ngrp, group, 0)

        for b in range(GRP):
            wait_scatter(b)

    return body(weight, idx)


def kernel(input_positions, weight):
    bsz, slen = input_positions.shape
    dim = weight.shape[1]
    total = bsz * slen
    nch = total // (NW * CH)
    idx = input_positions.astype(jnp.int32).reshape(NW, nch, CH)
    out = _gather(weight, idx, nch, dim)
    return out.reshape(bsz, slen, dim)


# speedup vs baseline: 2.3794x; 1.0122x over previous
import functools

import jax
import jax.numpy as jnp
from jax import lax
from jax.experimental import pallas as pl
from jax.experimental.pallas import tpu as pltpu
from jax.experimental.pallas import tpu_sc as plsc

NC = 2
NS = 16
NW = NC * NS
CH = 80
GRP = 4
V = 8192


@functools.partial(jax.jit, static_argnums=(2, 3))
def _gather(weight, idx, nch, dim):
    bpw = nch * CH
    nchh = nch // 2
    ngrp = nchh // GRP
    rpt = V // NS

    mesh = plsc.VectorSubcoreMesh(core_axis_name="c", subcore_axis_name="s")

    @functools.partial(
        pl.kernel,
        mesh=mesh,
        out_type=jax.ShapeDtypeStruct((NW * bpw, dim), jnp.float32),
        scratch_types=[
            pltpu.VMEM((nchh, CH), jnp.int32),
            pltpu.VMEM((GRP * CH, dim), jnp.float32),
            pltpu.VMEM_SHARED((V, dim), jnp.float32),
            pltpu.SemaphoreType.DMA,
            pltpu.SemaphoreType.DMA,
        ],
    )
    def body(table_hbm, idx_hbm, out_hbm, idx_v, rows_v, tab_s, gsem, ssem):
        c = lax.axis_index("c")
        s = lax.axis_index("s")
        wid = s * NC + c
        base = wid * bpw

        src0 = s * rpt

        @pl.when(s == 0)
        def _first_stripe():
            pltpu.sync_copy(table_hbm.at[pl.ds(8, rpt - 8)], tab_s.at[pl.ds(7, rpt - 8)])
            pltpu.sync_copy(table_hbm.at[pl.ds(0, 8)], rows_v.at[pl.ds(0, 8)])
            pltpu.sync_copy(rows_v.at[pl.ds(1, 7)], tab_s.at[pl.ds(0, 7)])

        @pl.when(s != 0)
        def _stripe():
            pltpu.sync_copy(table_hbm.at[pl.ds(src0, rpt)], tab_s.at[pl.ds(src0 - 1, rpt)])

        @pl.when(s == NS - 1)
        def _last_row():
            pltpu.sync_copy(table_hbm.at[pl.ds(V, 1)], rows_v.at[pl.ds(0, 1)])
            pltpu.sync_copy(rows_v.at[pl.ds(0, 1)], tab_s.at[pl.ds(V - 1, 1)])

        plsc.subcore_barrier()

        def start_gather(j, b):
            pltpu.async_copy(tab_s.at[idx_v.at[j]], rows_v.at[pl.ds(b * CH, CH)], gsem)

        def start_scatter(jg, b):
            pltpu.async_copy(
                rows_v.at[pl.ds(b * CH, CH)], out_hbm.at[pl.ds(base + jg * CH, CH)], ssem
            )

        def wait_gather(b):
            pltpu.make_async_copy(
                tab_s.at[pl.ds(0, CH)], rows_v.at[pl.ds(b * CH, CH)], gsem
            ).wait()

        def wait_scatter(b):
            pltpu.make_async_copy(
                rows_v.at[pl.ds(b * CH, CH)], out_hbm.at[pl.ds(0, CH)], ssem
            ).wait()

        for h in range(2):
            pltpu.sync_copy(idx_hbm.at[wid].at[h], idx_v)

            for b in range(GRP):
                if h > 0:
                    wait_scatter(b)
                start_gather(b, b)
            for b in range(GRP):
                wait_gather(b)
                start_scatter(h * nchh + b, b)

            def group(g, carry):
                j0 = g * GRP
                for b in range(GRP):
                    wait_scatter(b)
                    start_gather(j0 + b, b)
                for b in range(GRP):
                    wait_gather(b)
                    start_scatter(h * nchh + j0 + b, b)
                return carry

            lax.fori_loop(1, ngrp, group, 0)

        for b in range(GRP):
            wait_scatter(b)

    return body(weight, idx)


def kernel(input_positions, weight):
    bsz, slen = input_positions.shape
    dim = weight.shape[1]
    total = bsz * slen
    nch = total // (NW * CH)
    idx = input_positions.astype(jnp.int32).reshape(NW, 2, nch // 2, CH)
    out = _gather(weight, idx, nch, dim)
    return out.reshape(bsz, slen, dim)
